# Initial kernel scaffold; baseline (speedup 1.0000x reference)
import jax
import jax.numpy as jnp
from jax.experimental import pallas as pl


def kernel(feature, W1, b1, W2, b2, W3, b3):
    raise NotImplementedError("write your pallas kernel here")


# speedup vs baseline: 1.0055x; 1.0055x over previous
import functools

import jax
import jax.numpy as jnp
from jax.experimental import pallas as pl
from jax.experimental.pallas import tpu as pltpu

_VMEM_LIMIT = 110 * 1024 * 1024


def _pick_block(n):
    for b in (256, 200, 128, 80, 64, 40, 16, 8):
        if n % b == 0:
            return b
    return n


def _dist_kernel(xs_ref, xf_ref, d2_ref, max_ref, x2f_scr, *, blk, n):
    i = pl.program_id(0)
    xf = xf_ref[...]

    @pl.when(i == 0)
    def _():
        ones = jnp.ones((1, xf.shape[1]), jnp.float32)
        x2f_scr[...] = jax.lax.dot_general(
            ones, xf * xf, (((1,), (1,)), ((), ())),
            preferred_element_type=jnp.float32)

    xs = xs_ref[...]
    x2s = jnp.sum(xs * xs, axis=1, keepdims=True)
    g = jax.lax.dot_general(
        xs, xf, (((1,), (1,)), ((), ())), preferred_element_type=jnp.float32)
    d2 = jnp.maximum(x2s + x2f_scr[...] - 2.0 * g, 0.0)
    d2_ref[...] = d2
    row = i * blk + jax.lax.broadcasted_iota(jnp.int32, (blk, n), 0)
    col = jax.lax.broadcasted_iota(jnp.int32, (blk, n), 1)
    m = jnp.max(jnp.where(col > row, d2, -1.0))

    @pl.when(i == 0)
    def _():
        max_ref[0, 0] = m

    @pl.when(i > 0)
    def _():
        max_ref[0, 0] = jnp.maximum(max_ref[0, 0], m)


def _thresh_kernel(maxv_ref, d2_ref, mask_ref, deg_ref, dinv_ref, *, blk, n):
    i = pl.program_id(0)
    t = 0.5 * maxv_ref[0, 0]
    d2 = d2_ref[...]
    row = i * blk + jax.lax.broadcasted_iota(jnp.int32, (blk, n), 0)
    col = jax.lax.broadcasted_iota(jnp.int32, (blk, n), 1)
    m = jnp.logical_and(d2 < t, col > row)
    mask_ref[...] = m.astype(jnp.int8)
    colsum = jnp.sum(m.astype(jnp.float32), axis=0, keepdims=True)

    @pl.when(i == 0)
    def _():
        deg_ref[...] = 1.0 + colsum

    @pl.when(i > 0)
    def _():
        deg_ref[...] = deg_ref[...] + colsum

    @pl.when(i == pl.num_programs(0) - 1)
    def _():
        dinv_ref[...] = jax.lax.rsqrt(deg_ref[...])


def _conv_kernel(x_ref, w_ref, b_ref, dinv_ref, mask_ref,
                 xout_ref, pool_ref, v_scr, y_scr, *, blk, n):
    i = pl.program_id(0)
    xw = jnp.dot(x_ref[...], w_ref[...], preferred_element_type=jnp.float32)
    dstripe = dinv_ref[pl.ds(i * blk, blk), :]
    v = dstripe * xw
    v_scr[pl.ds(i * blk, blk), :] = v

    @pl.when(i == 0)
    def _():
        y_scr[...] = jnp.zeros_like(y_scr)

    maskf = mask_ref[...].astype(jnp.float32)
    y_scr[...] += jax.lax.dot_general(
        maskf, v, (((0,), (0,)), ((), ())), preferred_element_type=jnp.float32)

    @pl.when(i == pl.num_programs(0) - 1)
    def _():
        outx = dinv_ref[...] * (v_scr[...] + y_scr[...]) + b_ref[...]
        outx = jnp.maximum(outx, 0.0)
        xout_ref[...] = outx
        mx = jnp.max(outx, axis=0, keepdims=True)
        mean = jnp.sum(outx, axis=0, keepdims=True) * (1.0 / n)
        pool_ref[...] = jnp.concatenate([mx, mean], axis=1)


def _dist_call(x, blk):
    n, d = x.shape
    grid = (n // blk,)
    return pl.pallas_call(
        functools.partial(_dist_kernel, blk=blk, n=n),
        grid=grid,
        in_specs=[
            pl.BlockSpec((blk, d), lambda i: (i, 0)),
            pl.BlockSpec((n, d), lambda i: (0, 0)),
        ],
        out_specs=[
            pl.BlockSpec((blk, n), lambda i: (i, 0)),
            pl.BlockSpec(memory_space=pltpu.SMEM),
        ],
        out_shape=[
            jax.ShapeDtypeStruct((n, n), jnp.float32),
            jax.ShapeDtypeStruct((1, 1), jnp.float32),
        ],
        scratch_shapes=[pltpu.VMEM((1, n), jnp.float32)],
        compiler_params=pltpu.CompilerParams(
            vmem_limit_bytes=_VMEM_LIMIT),
    )(x, x)


def _thresh_call(maxv, d2, blk):
    n = d2.shape[0]
    grid = (n // blk,)
    return pl.pallas_call(
        functools.partial(_thresh_kernel, blk=blk, n=n),
        grid=grid,
        in_specs=[
            pl.BlockSpec(memory_space=pltpu.SMEM),
            pl.BlockSpec((blk, n), lambda i: (i, 0)),
        ],
        out_specs=[
            pl.BlockSpec((blk, n), lambda i: (i, 0)),
            pl.BlockSpec((1, n), lambda i: (0, 0)),
            pl.BlockSpec((1, n), lambda i: (0, 0)),
        ],
        out_shape=[
            jax.ShapeDtypeStruct((n, n), jnp.int8),
            jax.ShapeDtypeStruct((1, n), jnp.float32),
            jax.ShapeDtypeStruct((1, n), jnp.float32),
        ],
        compiler_params=pltpu.CompilerParams(
            vmem_limit_bytes=_VMEM_LIMIT),
    )(maxv, d2)


def _conv_call(x, w, b2d, dinv_col, mask, blk):
    n, din = x.shape
    h = w.shape[1]
    grid = (n // blk,)
    return pl.pallas_call(
        functools.partial(_conv_kernel, blk=blk, n=n),
        grid=grid,
        in_specs=[
            pl.BlockSpec((blk, din), lambda i: (i, 0)),
            pl.BlockSpec((din, h), lambda i: (0, 0)),
            pl.BlockSpec((1, h), lambda i: (0, 0)),
            pl.BlockSpec((n, 1), lambda i: (0, 0)),
            pl.BlockSpec((blk, n), lambda i: (i, 0)),
        ],
        out_specs=[
            pl.BlockSpec((n, h), lambda i: (0, 0)),
            pl.BlockSpec((1, 2 * h), lambda i: (0, 0)),
        ],
        out_shape=[
            jax.ShapeDtypeStruct((n, h), jnp.float32),
            jax.ShapeDtypeStruct((1, 2 * h), jnp.float32),
        ],
        scratch_shapes=[
            pltpu.VMEM((n, h), jnp.float32),
            pltpu.VMEM((n, h), jnp.float32),
        ],
        compiler_params=pltpu.CompilerParams(
            vmem_limit_bytes=_VMEM_LIMIT),
    )(x, w, b2d, dinv_col, mask)


def kernel(feature, W1, b1, W2, b2, W3, b3):
    n = feature.shape[0]
    blk = _pick_block(n)
    d2, maxv = _dist_call(feature, blk)
    mask, _deg, dinv_row = _thresh_call(maxv, d2, blk)
    dinv_col = dinv_row.reshape(n, 1)

    x = feature
    pooled = None
    for w, b in ((W1, b1), (W2, b2), (W3, b3)):
        x, p = _conv_call(x, w, b.reshape(1, -1), dinv_col, mask, blk)
        pooled = p if pooled is None else pooled + p
    return pooled


# speedup vs baseline: 1.3201x; 1.3128x over previous
import functools

import jax
import jax.numpy as jnp
from jax.experimental import pallas as pl
from jax.experimental.pallas import tpu as pltpu

_VMEM_LIMIT = 110 * 1024 * 1024


def _pick_block(n):
    for b in (256, 200, 128, 80, 64, 40, 16, 8):
        if n % b == 0:
            return b
    return n


def _dist_kernel(xs_ref, xf_ref, d2_ref, max_ref, x2f_scr, *, blk, n):
    i = pl.program_id(0)
    xf = xf_ref[...]

    @pl.when(i == 0)
    def _():
        ones = jnp.ones((1, xf.shape[1]), jnp.float32)
        x2f_scr[...] = jax.lax.dot_general(
            ones, xf * xf, (((1,), (1,)), ((), ())),
            preferred_element_type=jnp.float32)

    xs = xs_ref[...]
    x2s = jnp.sum(xs * xs, axis=1, keepdims=True)
    g = jax.lax.dot_general(
        xs, xf, (((1,), (1,)), ((), ())), preferred_element_type=jnp.float32)
    d2 = jnp.maximum(x2s + x2f_scr[...] - 2.0 * g, 0.0)
    d2_ref[...] = d2
    row = i * blk + jax.lax.broadcasted_iota(jnp.int32, (blk, n), 0)
    col = jax.lax.broadcasted_iota(jnp.int32, (blk, n), 1)
    m = jnp.max(jnp.where(col > row, d2, -1.0))

    @pl.when(i == 0)
    def _():
        max_ref[0, 0] = m

    @pl.when(i > 0)
    def _():
        max_ref[0, 0] = jnp.maximum(max_ref[0, 0], m)


def _thresh_kernel(maxv_ref, d2_ref, mask_ref, deg_ref, dinv_ref, cnt_ref,
                   *, blk, n):
    i = pl.program_id(0)
    t = 0.5 * maxv_ref[0, 0]
    d2 = d2_ref[...]
    row = i * blk + jax.lax.broadcasted_iota(jnp.int32, (blk, n), 0)
    col = jax.lax.broadcasted_iota(jnp.int32, (blk, n), 1)
    m = jnp.logical_and(d2 < t, col > row)
    mask_ref[...] = m.astype(jnp.int8)
    colsum = jnp.sum(m.astype(jnp.float32), axis=0, keepdims=True)
    cnt_ref[0, 0, 0] = jnp.sum(colsum).astype(jnp.int32)

    @pl.when(i == 0)
    def _():
        deg_ref[...] = 1.0 + colsum

    @pl.when(i > 0)
    def _():
        deg_ref[...] = deg_ref[...] + colsum

    @pl.when(i == pl.num_programs(0) - 1)
    def _():
        dinv_ref[...] = jax.lax.rsqrt(deg_ref[...])


def _conv_kernel(cnt_ref, midx_ref, x_ref, w_ref, b_ref, dinv_ref, mask_ref,
                 xout_ref, pool_ref, v_scr, y_scr, *, blk, n):
    i = pl.program_id(0)
    xw = jnp.dot(x_ref[...], w_ref[...], preferred_element_type=jnp.float32)
    dstripe = dinv_ref[pl.ds(i * blk, blk), :]
    v = dstripe * xw
    v_scr[pl.ds(i * blk, blk), :] = v

    @pl.when(i == 0)
    def _():
        y_scr[...] = jnp.zeros_like(y_scr)

    @pl.when(cnt_ref[i] > 0)
    def _():
        maskf = mask_ref[...].astype(jnp.float32)
        y_scr[...] += jax.lax.dot_general(
            maskf, v, (((0,), (0,)), ((), ())),
            preferred_element_type=jnp.float32)

    @pl.when(i == pl.num_programs(0) - 1)
    def _():
        outx = dinv_ref[...] * (v_scr[...] + y_scr[...]) + b_ref[...]
        outx = jnp.maximum(outx, 0.0)
        xout_ref[...] = outx
        mx = jnp.max(outx, axis=0, keepdims=True)
        mean = jnp.sum(outx, axis=0, keepdims=True) * (1.0 / n)
        pool_ref[...] = jnp.concatenate([mx, mean], axis=1)


def _dist_call(x, blk):
    n, d = x.shape
    grid = (n // blk,)
    return pl.pallas_call(
        functools.partial(_dist_kernel, blk=blk, n=n),
        grid=grid,
        in_specs=[
            pl.BlockSpec((blk, d), lambda i: (i, 0)),
            pl.BlockSpec((n, d), lambda i: (0, 0)),
        ],
        out_specs=[
            pl.BlockSpec((blk, n), lambda i: (i, 0)),
            pl.BlockSpec(memory_space=pltpu.SMEM),
        ],
        out_shape=[
            jax.ShapeDtypeStruct((n, n), jnp.float32),
            jax.ShapeDtypeStruct((1, 1), jnp.float32),
        ],
        scratch_shapes=[pltpu.VMEM((1, n), jnp.float32)],
        compiler_params=pltpu.CompilerParams(
            vmem_limit_bytes=_VMEM_LIMIT),
    )(x, x)


def _thresh_call(maxv, d2, blk):
    n = d2.shape[0]
    grid = (n // blk,)
    return pl.pallas_call(
        functools.partial(_thresh_kernel, blk=blk, n=n),
        grid=grid,
        in_specs=[
            pl.BlockSpec(memory_space=pltpu.SMEM),
            pl.BlockSpec((blk, n), lambda i: (i, 0)),
        ],
        out_specs=[
            pl.BlockSpec((blk, n), lambda i: (i, 0)),
            pl.BlockSpec((1, n), lambda i: (0, 0)),
            pl.BlockSpec((1, n), lambda i: (0, 0)),
            pl.BlockSpec((1, 1, 1), lambda i: (i, 0, 0),
                         memory_space=pltpu.SMEM),
        ],
        out_shape=[
            jax.ShapeDtypeStruct((n, n), jnp.int8),
            jax.ShapeDtypeStruct((1, n), jnp.float32),
            jax.ShapeDtypeStruct((1, n), jnp.float32),
            jax.ShapeDtypeStruct((n // blk, 1, 1), jnp.int32),
        ],
        compiler_params=pltpu.CompilerParams(
            vmem_limit_bytes=_VMEM_LIMIT),
    )(maxv, d2)


def _conv_call(cnt, midx, x, w, b2d, dinv_col, mask, blk):
    n, din = x.shape
    h = w.shape[1]
    grid_spec = pltpu.PrefetchScalarGridSpec(
        num_scalar_prefetch=2,
        grid=(n // blk,),
        in_specs=[
            pl.BlockSpec((blk, din), lambda i, c, m: (i, 0)),
            pl.BlockSpec((din, h), lambda i, c, m: (0, 0)),
            pl.BlockSpec((1, h), lambda i, c, m: (0, 0)),
            pl.BlockSpec((n, 1), lambda i, c, m: (0, 0)),
            pl.BlockSpec((blk, n), lambda i, c, m: (m[i], 0)),
        ],
        out_specs=[
            pl.BlockSpec((n, h), lambda i, c, m: (0, 0)),
            pl.BlockSpec((1, 2 * h), lambda i, c, m: (0, 0)),
        ],
        scratch_shapes=[
            pltpu.VMEM((n, h), jnp.float32),
            pltpu.VMEM((n, h), jnp.float32),
        ],
    )
    return pl.pallas_call(
        functools.partial(_conv_kernel, blk=blk, n=n),
        grid_spec=grid_spec,
        out_shape=[
            jax.ShapeDtypeStruct((n, h), jnp.float32),
            jax.ShapeDtypeStruct((1, 2 * h), jnp.float32),
        ],
        compiler_params=pltpu.CompilerParams(
            vmem_limit_bytes=_VMEM_LIMIT),
    )(cnt, midx, x, w, b2d, dinv_col, mask)


def kernel(feature, W1, b1, W2, b2, W3, b3):
    n = feature.shape[0]
    blk = _pick_block(n)
    d2, maxv = _dist_call(feature, blk)
    mask, _deg, dinv_row, cnt2d = _thresh_call(maxv, d2, blk)
    dinv_col = dinv_row.reshape(n, 1)
    cnt = cnt2d.reshape(-1)
    nblk = cnt.shape[0]
    midx = jnp.maximum(
        jax.lax.associative_scan(
            jnp.maximum, jnp.where(cnt > 0, jnp.arange(nblk, dtype=jnp.int32),
                                   jnp.int32(-1))),
        0)

    x = feature
    pooled = None
    for w, b in ((W1, b1), (W2, b2), (W3, b3)):
        x, p = _conv_call(cnt, midx, x, w, b.reshape(1, -1), dinv_col, mask,
                          blk)
        pooled = p if pooled is None else pooled + p
    return pooled


# speedup vs baseline: 1.3407x; 1.0156x over previous
import functools

import jax
import jax.numpy as jnp
from jax.experimental import pallas as pl
from jax.experimental.pallas import tpu as pltpu

_VMEM_LIMIT = 110 * 1024 * 1024


def _pick_block(n):
    for b in (256, 200, 128, 80, 64, 40, 16, 8):
        if n % b == 0:
            return b
    return n


def _dist_kernel(xs_ref, xf_ref, d2_ref, max_ref, x2f_scr, *, blk, n):
    i = pl.program_id(0)
    xf = xf_ref[...]

    @pl.when(i == 0)
    def _():
        ones = jnp.ones((1, xf.shape[1]), jnp.float32)
        x2f_scr[...] = jax.lax.dot_general(
            ones, xf * xf, (((1,), (1,)), ((), ())),
            preferred_element_type=jnp.float32)

    xs = xs_ref[...]
    x2s = jnp.sum(xs * xs, axis=1, keepdims=True)
    g = jax.lax.dot_general(
        xs, xf, (((1,), (1,)), ((), ())), preferred_element_type=jnp.float32)
    d2 = jnp.maximum(x2s + x2f_scr[...] - 2.0 * g, 0.0)
    row = i * blk + jax.lax.broadcasted_iota(jnp.int32, (blk, n), 0)
    col = jax.lax.broadcasted_iota(jnp.int32, (blk, n), 1)
    upper = col > row
    d2_ref[...] = jnp.where(upper, d2, jnp.inf)
    m = jnp.max(jnp.where(upper, d2, -1.0))

    @pl.when(i == 0)
    def _():
        max_ref[0, 0] = m

    @pl.when(i > 0)
    def _():
        max_ref[0, 0] = jnp.maximum(max_ref[0, 0], m)


def _thresh_kernel(maxv_ref, d2_ref, mask_ref, deg_ref, dinv_ref, cnt_ref,
                   *, blk, n):
    i = pl.program_id(0)
    t = 0.5 * maxv_ref[0, 0]
    m = d2_ref[...] < t
    mask_ref[...] = m.astype(jnp.int8)
    colsum = jnp.sum(m.astype(jnp.float32), axis=0, keepdims=True)
    cnt_ref[0, 0, 0] = jnp.sum(colsum).astype(jnp.int32)

    @pl.when(i == 0)
    def _():
        deg_ref[...] = 1.0 + colsum

    @pl.when(i > 0)
    def _():
        deg_ref[...] = deg_ref[...] + colsum

    @pl.when(i == pl.num_programs(0) - 1)
    def _():
        dinv_ref[...] = jax.lax.rsqrt(deg_ref[...])


def _conv_kernel(cnt_ref, midx_ref, x_ref, w_ref, b_ref, dinv_ref, mask_ref,
                 xout_ref, pool_ref, y_scr, mx_scr, sm_scr, *, blk, n):
    i = pl.program_id(0)
    xw = jnp.dot(x_ref[...], w_ref[...], preferred_element_type=jnp.float32)
    dstripe = dinv_ref[pl.ds(i * blk, blk), :]
    v = dstripe * xw

    @pl.when(i == 0)
    def _():
        y_scr[...] = jnp.zeros_like(y_scr)

    @pl.when(cnt_ref[i] > 0)
    def _():
        maskf = mask_ref[...].astype(jnp.float32)
        y_scr[...] += jax.lax.dot_general(
            maskf, v, (((0,), (0,)), ((), ())),
            preferred_element_type=jnp.float32)

    yi = y_scr[pl.ds(i * blk, blk), :]
    outx = jnp.maximum(dstripe * (v + yi) + b_ref[...], 0.0)
    xout_ref[...] = outx
    pmx = jnp.max(outx, axis=0, keepdims=True)
    psm = jnp.sum(outx, axis=0, keepdims=True)

    @pl.when(i == 0)
    def _():
        mx_scr[...] = pmx
        sm_scr[...] = psm

    @pl.when(i > 0)
    def _():
        mx_scr[...] = jnp.maximum(mx_scr[...], pmx)
        sm_scr[...] = sm_scr[...] + psm

    @pl.when(i == pl.num_programs(0) - 1)
    def _():
        pool_ref[...] = jnp.concatenate(
            [mx_scr[...], sm_scr[...] * (1.0 / n)], axis=1)


def _dist_call(x, blk):
    n, d = x.shape
    grid = (n // blk,)
    return pl.pallas_call(
        functools.partial(_dist_kernel, blk=blk, n=n),
        grid=grid,
        in_specs=[
            pl.BlockSpec((blk, d), lambda i: (i, 0)),
            pl.BlockSpec((n, d), lambda i: (0, 0)),
        ],
        out_specs=[
            pl.BlockSpec((blk, n), lambda i: (i, 0)),
            pl.BlockSpec(memory_space=pltpu.SMEM),
        ],
        out_shape=[
            jax.ShapeDtypeStruct((n, n), jnp.float32),
            jax.ShapeDtypeStruct((1, 1), jnp.float32),
        ],
        scratch_shapes=[pltpu.VMEM((1, n), jnp.float32)],
        compiler_params=pltpu.CompilerParams(
            vmem_limit_bytes=_VMEM_LIMIT),
    )(x, x)


def _thresh_call(maxv, d2, blk):
    n = d2.shape[0]
    grid = (n // blk,)
    return pl.pallas_call(
        functools.partial(_thresh_kernel, blk=blk, n=n),
        grid=grid,
        in_specs=[
            pl.BlockSpec(memory_space=pltpu.SMEM),
            pl.BlockSpec((blk, n), lambda i: (i, 0)),
        ],
        out_specs=[
            pl.BlockSpec((blk, n), lambda i: (i, 0)),
            pl.BlockSpec((1, n), lambda i: (0, 0)),
            pl.BlockSpec((1, n), lambda i: (0, 0)),
            pl.BlockSpec((1, 1, 1), lambda i: (i, 0, 0),
                         memory_space=pltpu.SMEM),
        ],
        out_shape=[
            jax.ShapeDtypeStruct((n, n), jnp.int8),
            jax.ShapeDtypeStruct((1, n), jnp.float32),
            jax.ShapeDtypeStruct((1, n), jnp.float32),
            jax.ShapeDtypeStruct((n // blk, 1, 1), jnp.int32),
        ],
        compiler_params=pltpu.CompilerParams(
            vmem_limit_bytes=_VMEM_LIMIT),
    )(maxv, d2)


def _conv_call(cnt, midx, x, w, b2d, dinv_col, mask, blk):
    n, din = x.shape
    h = w.shape[1]
    grid_spec = pltpu.PrefetchScalarGridSpec(
        num_scalar_prefetch=2,
        grid=(n // blk,),
        in_specs=[
            pl.BlockSpec((blk, din), lambda i, c, m: (i, 0)),
            pl.BlockSpec((din, h), lambda i, c, m: (0, 0)),
            pl.BlockSpec((1, h), lambda i, c, m: (0, 0)),
            pl.BlockSpec((n, 1), lambda i, c, m: (0, 0)),
            pl.BlockSpec((blk, n), lambda i, c, m: (m[i], 0)),
        ],
        out_specs=[
            pl.BlockSpec((blk, h), lambda i, c, m: (i, 0)),
            pl.BlockSpec((1, 2 * h), lambda i, c, m: (0, 0)),
        ],
        scratch_shapes=[
            pltpu.VMEM((n, h), jnp.float32),
            pltpu.VMEM((1, h), jnp.float32),
            pltpu.VMEM((1, h), jnp.float32),
        ],
    )
    return pl.pallas_call(
        functools.partial(_conv_kernel, blk=blk, n=n),
        grid_spec=grid_spec,
        out_shape=[
            jax.ShapeDtypeStruct((n, h), jnp.float32),
            jax.ShapeDtypeStruct((1, 2 * h), jnp.float32),
        ],
        compiler_params=pltpu.CompilerParams(
            vmem_limit_bytes=_VMEM_LIMIT),
    )(cnt, midx, x, w, b2d, dinv_col, mask)


def kernel(feature, W1, b1, W2, b2, W3, b3):
    n = feature.shape[0]
    blk = _pick_block(n)
    d2, maxv = _dist_call(feature, blk)
    mask, _deg, dinv_row, cnt2d = _thresh_call(maxv, d2, blk)
    dinv_col = dinv_row.reshape(n, 1)
    cnt = cnt2d.reshape(-1)
    nblk = cnt.shape[0]
    midx = jnp.maximum(
        jax.lax.associative_scan(
            jnp.maximum, jnp.where(cnt > 0, jnp.arange(nblk, dtype=jnp.int32),
                                   jnp.int32(-1))),
        0)

    x = feature
    pooled = None
    for w, b in ((W1, b1), (W2, b2), (W3, b3)):
        x, p = _conv_call(cnt, midx, x, w, b.reshape(1, -1), dinv_col, mask,
                          blk)
        pooled = p if pooled is None else pooled + p
    return pooled


# speedup vs baseline: 1.5506x; 1.1565x over previous
import functools

import jax
import jax.numpy as jnp
from jax.experimental import pallas as pl
from jax.experimental.pallas import tpu as pltpu

_VMEM_LIMIT = 63 * 1024 * 1024
_BI = 1000
_CJ = 2048


def _dist_kernel(xs_ref, xj_ref, d2_ref, max_ref, *, bi, cj, n):
    i = pl.program_id(0)
    j = pl.program_id(1)
    fl = (i * bi) // cj

    @pl.when(j >= fl)
    def _():
        xs = xs_ref[...]
        x2s = jnp.sum(xs * xs, axis=1, keepdims=True)
        ones = jnp.ones((1, xs.shape[1]), jnp.float32)
        half = cj // 2
        for h in range(2):
            xjh = xj_ref[h * half:(h + 1) * half, :]
            gh = jax.lax.dot_general(
                xs, xjh, (((1,), (1,)), ((), ())),
                preferred_element_type=jnp.float32)
            x2jh = jax.lax.dot_general(
                ones, xjh * xjh, (((1,), (1,)), ((), ())),
                preferred_element_type=jnp.float32)
            col = (j * cj + h * half
                   + jax.lax.broadcasted_iota(jnp.int32, (bi, half), 1))
            row = i * bi + jax.lax.broadcasted_iota(jnp.int32, (bi, half), 0)
            x2jh = jnp.where(col[0:1, :] < n, x2jh, jnp.inf)
            d2h = jnp.maximum(x2s + x2jh - 2.0 * gh, 0.0)
            upper = col > row
            d2_ref[:, h * half:(h + 1) * half] = jnp.where(upper, d2h, jnp.inf)
            mh = jnp.max(jnp.where(jnp.logical_and(upper, col < n), d2h, -1.0))

            @pl.when(jnp.logical_and(i == 0, j == 0))
            def _():
                if h == 0:
                    max_ref[0, 0] = mh
                else:
                    max_ref[0, 0] = jnp.maximum(max_ref[0, 0], mh)

            @pl.when(jnp.logical_or(i > 0, j > 0))
            def _():
                max_ref[0, 0] = jnp.maximum(max_ref[0, 0], mh)


def _thresh_kernel(maxv_ref, d2_ref, mask_ref, dinv_ref, cnt_ref, deg_scr,
                   *, bi, cj, nj):
    i = pl.program_id(0)
    j = pl.program_id(1)
    ni = pl.num_programs(0)
    fl = (i * bi) // cj
    t = 0.5 * maxv_ref[0, 0]

    @pl.when(j == 0)
    def _():
        cnt_ref[0, 0, 0] = 0

    @pl.when(j >= fl)
    def _():
        m = d2_ref[...] < t
        mask_ref[...] = m.astype(jnp.int8)
        mf = m.astype(jnp.float32)
        colsum = jnp.sum(mf, axis=0, keepdims=True)
        cnt_ref[0, 0, 0] += jnp.sum(colsum).astype(jnp.int32)

        @pl.when(i == 0)
        def _():
            deg_scr[j, :, :] = 1.0 + colsum

        @pl.when(i > 0)
        def _():
            deg_scr[j, :, :] = deg_scr[j, :, :] + colsum

    @pl.when(j < fl)
    def _():
        mask_ref[...] = jnp.zeros_like(mask_ref)

    @pl.when(i == ni - 1)
    def _():
        dinv_ref[...] = jax.lax.rsqrt(deg_scr[j, :, :])


def _conv_kernel(cnt_ref, midx_ref, x_ref, w_ref, b_ref, dinv_ref, mask_ref,
                 xout_ref, pool_ref, y_scr, mx_scr, sm_scr, *, blk, n):
    i = pl.program_id(0)
    xw = jnp.dot(x_ref[...], w_ref[...], preferred_element_type=jnp.float32)
    dstripe = dinv_ref[...]
    v = dstripe * xw

    @pl.when(i == 0)
    def _():
        y_scr[...] = jnp.zeros_like(y_scr)

    @pl.when(cnt_ref[i] > 0)
    def _():
        maskf = mask_ref[...].astype(jnp.float32)
        y_scr[...] += jax.lax.dot_general(
            maskf, v, (((0,), (0,)), ((), ())),
            preferred_element_type=jnp.float32)

    yi = y_scr[pl.ds(i * blk, blk), :]
    outx = jnp.maximum(dstripe * (v + yi) + b_ref[...], 0.0)
    xout_ref[...] = outx
    pmx = jnp.max(outx, axis=0, keepdims=True)
    psm = jnp.sum(outx, axis=0, keepdims=True)

    @pl.when(i == 0)
    def _():
        mx_scr[...] = pmx
        sm_scr[...] = psm

    @pl.when(i > 0)
    def _():
        mx_scr[...] = jnp.maximum(mx_scr[...], pmx)
        sm_scr[...] = sm_scr[...] + psm

    @pl.when(i == pl.num_programs(0) - 1)
    def _():
        pool_ref[...] = jnp.concatenate(
            [mx_scr[...], sm_scr[...] * (1.0 / n)], axis=1)


def _dist_call(x, xpad, bi, cj):
    n, d = x.shape
    npad = xpad.shape[0]
    ti, nj = n // bi, npad // cj

    def clamp(i, j):
        return (i, jnp.maximum(j, (i * bi) // cj))

    return pl.pallas_call(
        functools.partial(_dist_kernel, bi=bi, cj=cj, n=n),
        grid=(ti, nj),
        in_specs=[
            pl.BlockSpec((bi, d), lambda i, j: (i, 0)),
            pl.BlockSpec((cj, d), lambda i, j: clamp(i, j)[1:] + (0,)),
        ],
        out_specs=[
            pl.BlockSpec((bi, cj), clamp),
            pl.BlockSpec(memory_space=pltpu.SMEM),
        ],
        out_shape=[
            jax.ShapeDtypeStruct((n, npad), jnp.float32),
            jax.ShapeDtypeStruct((1, 1), jnp.float32),
        ],
        compiler_params=pltpu.CompilerParams(
            vmem_limit_bytes=_VMEM_LIMIT),
    )(x, xpad)


def _thresh_call(maxv, d2, bi, cj):
    n, npad = d2.shape
    ti, nj = n // bi, npad // cj

    def clamp(i, j):
        return (i, jnp.maximum(j, (i * bi) // cj))

    return pl.pallas_call(
        functools.partial(_thresh_kernel, bi=bi, cj=cj, nj=nj),
        grid=(ti, nj),
        in_specs=[
            pl.BlockSpec(memory_space=pltpu.SMEM),
            pl.BlockSpec((bi, cj), clamp),
        ],
        out_specs=[
            pl.BlockSpec((bi, cj), lambda i, j: (i, j)),
            pl.BlockSpec((1, cj), lambda i, j: (0, j)),
            pl.BlockSpec((1, 1, 1), lambda i, j: (i, 0, 0),
                         memory_space=pltpu.SMEM),
        ],
        out_shape=[
            jax.ShapeDtypeStruct((n, npad), jnp.int8),
            jax.ShapeDtypeStruct((1, npad), jnp.float32),
            jax.ShapeDtypeStruct((ti, 1, 1), jnp.int32),
        ],
        scratch_shapes=[pltpu.VMEM((nj, 1, cj), jnp.float32)],
        compiler_params=pltpu.CompilerParams(
            vmem_limit_bytes=_VMEM_LIMIT),
    )(maxv, d2)


def _conv_call(cnt, midx, x, w, b2d, dinv_col, mask, blk):
    n, din = x.shape
    npad = mask.shape[1]
    h = w.shape[1]
    grid_spec = pltpu.PrefetchScalarGridSpec(
        num_scalar_prefetch=2,
        grid=(n // blk,),
        in_specs=[
            pl.BlockSpec((blk, din), lambda i, c, m: (i, 0)),
            pl.BlockSpec((din, h), lambda i, c, m: (0, 0)),
            pl.BlockSpec((1, h), lambda i, c, m: (0, 0)),
            pl.BlockSpec((blk, 1), lambda i, c, m: (i, 0)),
            pl.BlockSpec((blk, npad), lambda i, c, m: (m[i], 0)),
        ],
        out_specs=[
            pl.BlockSpec((blk, h), lambda i, c, m: (i, 0)),
            pl.BlockSpec((1, 2 * h), lambda i, c, m: (0, 0)),
        ],
        scratch_shapes=[
            pltpu.VMEM((npad, h), jnp.float32),
            pltpu.VMEM((1, h), jnp.float32),
            pltpu.VMEM((1, h), jnp.float32),
        ],
    )
    return pl.pallas_call(
        functools.partial(_conv_kernel, blk=blk, n=n),
        grid_spec=grid_spec,
        out_shape=[
            jax.ShapeDtypeStruct((n, h), jnp.float32),
            jax.ShapeDtypeStruct((1, 2 * h), jnp.float32),
        ],
        compiler_params=pltpu.CompilerParams(
            vmem_limit_bytes=_VMEM_LIMIT),
    )(cnt, midx, x, w, b2d, dinv_col, mask)


def kernel(feature, W1, b1, W2, b2, W3, b3):
    n = feature.shape[0]
    if n % _BI == 0:
        bi, cj = _BI, _CJ
    else:
        bi, cj = n, 128
    npad = ((n + cj - 1) // cj) * cj
    xpad = jnp.pad(feature, ((0, npad - n), (0, 0)))
    d2, maxv = _dist_call(feature, xpad, bi, cj)
    mask, dinv_row, cnt3d = _thresh_call(maxv, d2, bi, cj)
    dinv_col = dinv_row[:, :n].reshape(n, 1)
    cnt = cnt3d.reshape(-1)
    nblk = cnt.shape[0]
    midx = jnp.maximum(
        jax.lax.associative_scan(
            jnp.maximum, jnp.where(cnt > 0, jnp.arange(nblk, dtype=jnp.int32),
                                   jnp.int32(-1))),
        0)

    x = feature
    pooled = None
    for w, b in ((W1, b1), (W2, b2), (W3, b3)):
        x, p = _conv_call(cnt, midx, x, w, b.reshape(1, -1), dinv_col, mask,
                          bi)
        pooled = p if pooled is None else pooled + p
    return pooled


# speedup vs baseline: 2.0472x; 1.3203x over previous
import functools

import jax
import jax.numpy as jnp
from jax.experimental import pallas as pl
from jax.experimental.pallas import tpu as pltpu

_VMEM_LIMIT = 63 * 1024 * 1024
_BI = 1000
_CJ = 2048


def _dist_kernel(xs_ref, xj_ref, d2_ref, max_ref, x2j_scr, x2s_scr,
                 *, bi, cj, n):
    i = pl.program_id(0)
    j = pl.program_id(1)
    fl = (i * bi) // cj

    @pl.when(i == 0)
    def _():
        xj = xj_ref[...]
        ones = jnp.ones((1, xj.shape[1]), jnp.float32)
        x2j = jax.lax.dot_general(
            ones, xj * xj, (((1,), (1,)), ((), ())),
            preferred_element_type=jnp.float32)
        colr = j * cj + jax.lax.broadcasted_iota(jnp.int32, (1, cj), 1)
        x2j_scr[j, :, :] = jnp.where(colr < n, x2j, -jnp.inf)

    @pl.when(j == fl)
    def _():
        xs = xs_ref[...]
        x2s_scr[...] = jnp.sum(xs * xs, axis=1, keepdims=True)

    mixed = j * cj < (i + 1) * bi

    def body(with_sel):
        xs = xs_ref[...]
        x2s = x2s_scr[...]
        quarter = cj // 4
        parts = []
        for h in range(4):
            xjh = xj_ref[h * quarter:(h + 1) * quarter, :]
            gh = jax.lax.dot_general(
                xs, xjh, (((1,), (1,)), ((), ())),
                preferred_element_type=jnp.float32)
            x2jh = x2j_scr[j, :, h * quarter:(h + 1) * quarter]
            d2h = jnp.maximum(x2s + x2jh - 2.0 * gh, 0.0)
            if with_sel:
                col = (j * cj + h * quarter
                       + jax.lax.broadcasted_iota(jnp.int32, (bi, quarter), 1))
                row = (i * bi
                       + jax.lax.broadcasted_iota(jnp.int32, (bi, quarter), 0))
                upper = col > row
                d2_ref[:, h * quarter:(h + 1) * quarter] = (
                    jnp.where(upper, d2h, jnp.inf))
                parts.append(jnp.max(jnp.where(upper, d2h, -1.0)))
            else:
                d2_ref[:, h * quarter:(h + 1) * quarter] = d2h
                parts.append(jnp.max(d2h))
        mh = jnp.maximum(jnp.maximum(parts[0], parts[1]),
                         jnp.maximum(parts[2], parts[3]))

        @pl.when(jnp.logical_and(i == 0, j == 0))
        def _():
            max_ref[0, 0] = mh

        @pl.when(jnp.logical_or(i > 0, j > 0))
        def _():
            max_ref[0, 0] = jnp.maximum(max_ref[0, 0], mh)

    @pl.when(jnp.logical_and(j >= fl, mixed))
    def _():
        body(True)

    @pl.when(jnp.logical_not(mixed))
    def _():
        body(False)


def _thresh_kernel(maxv_ref, d2_ref, mask_ref, dinv_ref, cnt_ref, deg_scr,
                   *, bi, cj, nj, n):
    i = pl.program_id(0)
    j = pl.program_id(1)
    ni = pl.num_programs(0)
    fl = (i * bi) // cj
    t = 0.5 * maxv_ref[0, 0]

    @pl.when(j == 0)
    def _():
        cnt_ref[0, 0, 0] = 0

    @pl.when(j >= fl)
    def _():
        m = d2_ref[...] < t
        mask_ref[...] = m.astype(jnp.int8)
        mf = m.astype(jnp.float32)
        colsum = jnp.sum(mf, axis=0, keepdims=True)
        colr = j * cj + jax.lax.broadcasted_iota(jnp.int32, (1, cj), 1)
        real = jnp.where(colr < n, colsum, 0.0)
        cnt_ref[0, 0, 0] += jnp.sum(real).astype(jnp.int32)

        @pl.when(i == 0)
        def _():
            deg_scr[j, :, :] = 1.0 + colsum

        @pl.when(i > 0)
        def _():
            deg_scr[j, :, :] = deg_scr[j, :, :] + colsum

    @pl.when(i == ni - 1)
    def _():
        dinv_ref[...] = jax.lax.rsqrt(deg_scr[j, :, :])


def _conv_kernel(cnt_ref, midx_ref, x_ref, w_ref, b_ref, dinv_ref, mask_ref,
                 xout_ref, pool_ref, y_scr, mx_scr, sm_scr, *, blk, n):
    i = pl.program_id(0)
    xw = jnp.dot(x_ref[...], w_ref[...], preferred_element_type=jnp.float32)
    dstripe = dinv_ref[...]
    v = dstripe * xw

    @pl.when(i == 0)
    def _():
        y_scr[...] = jnp.zeros_like(y_scr)

    @pl.when(cnt_ref[i] > 0)
    def _():
        maskf = mask_ref[...].astype(jnp.float32)
        y_scr[...] += jax.lax.dot_general(
            maskf, v, (((0,), (0,)), ((), ())),
            preferred_element_type=jnp.float32)

    yi = y_scr[pl.ds(i * blk, blk), :]
    outx = jnp.maximum(dstripe * (v + yi) + b_ref[...], 0.0)
    xout_ref[...] = outx
    pmx = jnp.max(outx, axis=0, keepdims=True)
    psm = jnp.sum(outx, axis=0, keepdims=True)

    @pl.when(i == 0)
    def _():
        mx_scr[...] = pmx
        sm_scr[...] = psm

    @pl.when(i > 0)
    def _():
        mx_scr[...] = jnp.maximum(mx_scr[...], pmx)
        sm_scr[...] = sm_scr[...] + psm

    @pl.when(i == pl.num_programs(0) - 1)
    def _():
        pool_ref[...] = jnp.concatenate(
            [mx_scr[...], sm_scr[...] * (1.0 / n)], axis=1)


def _dist_call(x, xpad, bi, cj):
    n, d = x.shape
    npad = xpad.shape[0]
    ti, nj = n // bi, npad // cj

    def clamp(i, j):
        return (i, jnp.maximum(j, (i * bi) // cj))

    return pl.pallas_call(
        functools.partial(_dist_kernel, bi=bi, cj=cj, n=n),
        grid=(ti, nj),
        in_specs=[
            pl.BlockSpec((bi, d), lambda i, j: (i, 0)),
            pl.BlockSpec((cj, d), lambda i, j: clamp(i, j)[1:] + (0,)),
        ],
        out_specs=[
            pl.BlockSpec((bi, cj), clamp),
            pl.BlockSpec(memory_space=pltpu.SMEM),
        ],
        out_shape=[
            jax.ShapeDtypeStruct((n, npad), jnp.float32),
            jax.ShapeDtypeStruct((1, 1), jnp.float32),
        ],
        scratch_shapes=[
            pltpu.VMEM((nj, 1, cj), jnp.float32),
            pltpu.VMEM((bi, 1), jnp.float32),
        ],
        compiler_params=pltpu.CompilerParams(
            vmem_limit_bytes=_VMEM_LIMIT),
    )(x, xpad)


def _thresh_call(maxv, d2, bi, cj):
    n, npad = d2.shape
    ti, nj = n // bi, npad // cj

    def clamp(i, j):
        return (i, jnp.maximum(j, (i * bi) // cj))

    return pl.pallas_call(
        functools.partial(_thresh_kernel, bi=bi, cj=cj, nj=nj, n=n),
        grid=(ti, nj),
        in_specs=[
            pl.BlockSpec(memory_space=pltpu.SMEM),
            pl.BlockSpec((bi, cj), clamp),
        ],
        out_specs=[
            pl.BlockSpec((bi, cj), clamp),
            pl.BlockSpec((1, cj), lambda i, j: (0, j)),
            pl.BlockSpec((1, 1, 1), lambda i, j: (i, 0, 0),
                         memory_space=pltpu.SMEM),
        ],
        out_shape=[
            jax.ShapeDtypeStruct((n, npad), jnp.int8),
            jax.ShapeDtypeStruct((1, npad), jnp.float32),
            jax.ShapeDtypeStruct((ti, 1, 1), jnp.int32),
        ],
        scratch_shapes=[pltpu.VMEM((nj, 1, cj), jnp.float32)],
        compiler_params=pltpu.CompilerParams(
            vmem_limit_bytes=_VMEM_LIMIT),
    )(maxv, d2)


def _conv_call(cnt, midx, x, w, b2d, dinv_col, mask, blk):
    n, din = x.shape
    npad = mask.shape[1]
    h = w.shape[1]
    grid_spec = pltpu.PrefetchScalarGridSpec(
        num_scalar_prefetch=2,
        grid=(n // blk,),
        in_specs=[
            pl.BlockSpec((blk, din), lambda i, c, m: (i, 0)),
            pl.BlockSpec((din, h), lambda i, c, m: (0, 0)),
            pl.BlockSpec((1, h), lambda i, c, m: (0, 0)),
            pl.BlockSpec((blk, 1), lambda i, c, m: (i, 0)),
            pl.BlockSpec((blk, npad), lambda i, c, m: (m[i], 0)),
        ],
        out_specs=[
            pl.BlockSpec((blk, h), lambda i, c, m: (i, 0)),
            pl.BlockSpec((1, 2 * h), lambda i, c, m: (0, 0)),
        ],
        scratch_shapes=[
            pltpu.VMEM((npad, h), jnp.float32),
            pltpu.VMEM((1, h), jnp.float32),
            pltpu.VMEM((1, h), jnp.float32),
        ],
    )
    return pl.pallas_call(
        functools.partial(_conv_kernel, blk=blk, n=n),
        grid_spec=grid_spec,
        out_shape=[
            jax.ShapeDtypeStruct((n, h), jnp.float32),
            jax.ShapeDtypeStruct((1, 2 * h), jnp.float32),
        ],
        compiler_params=pltpu.CompilerParams(
            vmem_limit_bytes=_VMEM_LIMIT),
    )(cnt, midx, x, w, b2d, dinv_col, mask)


def kernel(feature, W1, b1, W2, b2, W3, b3):
    n = feature.shape[0]
    if n % _BI == 0:
        bi, cj = _BI, _CJ
    else:
        bi, cj = n, 128
    npad = ((n + cj - 1) // cj) * cj
    xpad = jnp.pad(feature, ((0, npad - n), (0, 0)))
    d2, maxv = _dist_call(feature, xpad, bi, cj)
    mask, dinv_row, cnt3d = _thresh_call(maxv, d2, bi, cj)
    dinv_col = dinv_row[:, :n].reshape(n, 1)
    cnt = cnt3d.reshape(-1)
    nblk = cnt.shape[0]
    midx = jnp.maximum(
        jax.lax.associative_scan(
            jnp.maximum, jnp.where(cnt > 0, jnp.arange(nblk, dtype=jnp.int32),
                                   jnp.int32(-1))),
        0)

    x = feature
    pooled = None
    for w, b in ((W1, b1), (W2, b2), (W3, b3)):
        x, p = _conv_call(cnt, midx, x, w, b.reshape(1, -1), dinv_col, mask,
                          bi)
        pooled = p if pooled is None else pooled + p
    return pooled
